# baseline (speedup 1.0000x reference)
import functools

import jax
import jax.numpy as jnp
from jax import lax
from jax.experimental import pallas as pl
from jax.experimental.pallas import tpu as pltpu
from jax.experimental.pallas import tpu_sc as plsc

NC = 2
NS = 16
NW = NC * NS
CW = 128
D = 64
CHUNKS = 50
K = 5
NGRP = 3
T = CHUNKS // K
B = NW * CHUNKS * CW

_mesh = plsc.VectorSubcoreMesh(
    core_axis_name="c", subcore_axis_name="s", num_cores=NC, num_subcores=NS
)


@functools.partial(
    pl.kernel,
    out_type=jax.ShapeDtypeStruct((B, D), jnp.float32),
    mesh=_mesh,
    scratch_types=(
        [pltpu.VMEM((CHUNKS, CW), jnp.int32)]
        + [pltpu.VMEM((NGRP, K * CW, D), jnp.float32)]
        + [pltpu.SemaphoreType.DMA] * (1 + 2 * NGRP)
    ),
    compiler_params=pltpu.CompilerParams(use_tc_tiling_on_sc=False),
)
def _emb_lookup(table_hbm, idx_hbm, out_hbm, idx_v, grp_v, isem, *sems):
    gsems = sems[:NGRP]
    ssems = sems[NGRP:]
    wid = lax.axis_index("s") * NC + lax.axis_index("c")

    pltpu.async_copy(idx_hbm.at[wid], idx_v, isem).wait()

    base_row = wid * CHUNKS * CW

    def start_gathers(t, p):
        for j in range(K):
            pltpu.async_copy(
                table_hbm.at[idx_v.at[t * K + j]],
                grp_v.at[p].at[pl.ds(j * CW, CW)],
                gsems[p],
            )

    def drain_gathers(p):
        pltpu.make_async_copy(
            out_hbm.at[pl.ds(0, K * CW)], grp_v.at[p], gsems[p]
        ).wait()

    def start_scatter(t, p):
        pltpu.async_copy(
            grp_v.at[p],
            out_hbm.at[pl.ds(base_row + t * K * CW, K * CW)],
            ssems[p],
        )

    def wait_scatter(p):
        pltpu.make_async_copy(
            grp_v.at[p], out_hbm.at[pl.ds(0, K * CW)], ssems[p]
        ).wait()

    start_gathers(0, 0)
    for t in range(T):
        p = t % NGRP
        drain_gathers(p)
        start_scatter(t, p)
        f = t + 1
        if f < T:
            pf = f % NGRP
            if f - NGRP >= 0:
                wait_scatter(pf)
            start_gathers(f, pf)
    for t in range(T - NGRP, T):
        wait_scatter(t % NGRP)


def kernel(x, table):
    if x.ndim > 1 and x.shape[-1] == 1:
        x = x[..., 0]
    lead_shape = x.shape
    idx = x.reshape(NW, CHUNKS, CW).astype(jnp.int32)
    out = _emb_lookup(table.astype(jnp.float32), idx)
    return out.reshape(*lead_shape, D)


# speedup vs baseline: 1.7283x; 1.7283x over previous
import functools

import jax
import jax.numpy as jnp
from jax import lax
from jax.experimental import pallas as pl
from jax.experimental.pallas import tpu as pltpu
from jax.experimental.pallas import tpu_sc as plsc

NC = 2
NS = 16
NW = NC * NS
BATCH = 4096
HIST = 50
D = 64
V = 89
CW = 128
T = BATCH * HIST // 2 // (NW * CW)
NGRP = 3
NPAIR = BATCH * HIST // 2

_mesh = plsc.VectorSubcoreMesh(
    core_axis_name="c", subcore_axis_name="s", num_cores=NC, num_subcores=NS
)


@functools.partial(
    pl.kernel,
    out_type=jax.ShapeDtypeStruct((NPAIR, 2 * D), jnp.float32),
    mesh=_mesh,
    scratch_types=(
        [pltpu.VMEM((T, CW), jnp.int32)]
        + [pltpu.VMEM((NGRP, CW, 2 * D), jnp.float32)]
        + [pltpu.SemaphoreType.DMA] * (1 + 2 * NGRP)
    ),
    compiler_params=pltpu.CompilerParams(use_tc_tiling_on_sc=False),
)
def _emb_lookup(ptab_hbm, pidx_hbm, out_hbm, idx_v, grp_v, isem, *sems):
    gsems = sems[:NGRP]
    ssems = sems[NGRP:]
    wid = lax.axis_index("s") * NC + lax.axis_index("c")

    pltpu.async_copy(pidx_hbm.at[wid], idx_v, isem).wait()

    base = wid * T * CW

    def start_gather(t, p):
        pltpu.async_copy(ptab_hbm.at[idx_v.at[t]], grp_v.at[p], gsems[p])

    def wait_gather(p):
        pltpu.make_async_copy(
            out_hbm.at[pl.ds(0, CW)], grp_v.at[p], gsems[p]
        ).wait()

    def start_scatter(t, p):
        pltpu.async_copy(
            grp_v.at[p], out_hbm.at[pl.ds(base + t * CW, CW)], ssems[p]
        )

    def wait_scatter(p):
        pltpu.make_async_copy(
            grp_v.at[p], out_hbm.at[pl.ds(0, CW)], ssems[p]
        ).wait()

    start_gather(0, 0)
    for t in range(T):
        p = t % NGRP
        wait_gather(p)
        start_scatter(t, p)
        f = t + 1
        if f < T:
            pf = f % NGRP
            if f - NGRP >= 0:
                wait_scatter(pf)
            start_gather(f, pf)
    for t in range(T - NGRP, T):
        wait_scatter(t % NGRP)


def kernel(x, table):
    if x.ndim > 1 and x.shape[-1] == 1:
        x = x[..., 0]
    table = table.astype(jnp.float32)
    ptab = jnp.concatenate(
        [
            jnp.broadcast_to(table[:, None, :], (V, V, D)),
            jnp.broadcast_to(table[None, :, :], (V, V, D)),
        ],
        axis=-1,
    ).reshape(V * V, 2 * D)
    xi = x.astype(jnp.int32)
    pidx = (xi[:, 0::2] * V + xi[:, 1::2]).reshape(NW, T, CW)
    out = _emb_lookup(ptab, pidx)
    return out.reshape(BATCH, HIST, D)


# speedup vs baseline: 2.8093x; 1.6254x over previous
import functools

import jax
import jax.numpy as jnp
from jax import lax
from jax.experimental import pallas as pl
from jax.experimental.pallas import tpu as pltpu
from jax.experimental.pallas import tpu_sc as plsc

NC = 2
NS = 16
NW = NC * NS
BATCH = 4096
HIST = 50
D = 64
V = 89
IPT = BATCH // NW
NGRP = 3

_mesh = plsc.VectorSubcoreMesh(
    core_axis_name="c", subcore_axis_name="s", num_cores=NC, num_subcores=NS
)


@functools.partial(
    pl.kernel,
    out_type=jax.ShapeDtypeStruct((HIST, D // 8, NW, 8, IPT), jnp.float32),
    mesh=_mesh,
    scratch_types=(
        [pltpu.VMEM((HIST, IPT), jnp.int32)]
        + [pltpu.VMEM((NGRP, IPT, D), jnp.float32)]
        + [pltpu.VMEM((NGRP, D // 8, 8, IPT), jnp.float32)]
        + [pltpu.SemaphoreType.DMA] * (1 + 2 * NGRP)
    ),
    compiler_params=pltpu.CompilerParams(
        use_tc_tiling_on_sc=False, needs_layout_passes=False
    ),
)
def _emb_lookup(trep_hbm, idx_hbm, out_hbm, idx_v, gbuf, tbuf, isem, *sems):
    gsems = sems[:NGRP]
    ssems = sems[NGRP:]
    wid = lax.axis_index("s") * NC + lax.axis_index("c")

    pltpu.async_copy(idx_hbm.at[wid], idx_v, isem).wait()

    tab = trep_hbm.at[wid]

    def gather_desc(h, p):
        return pltpu.make_async_copy(tab.at[idx_v.at[h]], gbuf.at[p], gsems[p])

    def scatter_desc(h, p):
        return pltpu.make_async_copy(
            tbuf.at[p], out_hbm.at[h, :, wid], ssems[p]
        )

    iota = lax.iota(jnp.int32, 16)
    rot = [(iota + k) & 15 for k in range(16)]

    def transpose_chunk(p):
        gsrc = gbuf.at[p]
        tdst = tbuf.at[p]

        def blk(i, c):
            b0 = (i // 4) * 16
            d0 = (i % 4) * 16
            brows = b0 + iota
            for k in range(16):
                cols = d0 + rot[k]
                v = plsc.load_gather(gsrc, [brows, cols])
                plsc.store_scatter(tdst, [cols >> 3, cols & 7, brows], v)
            return c

        lax.fori_loop(0, 32, blk, 0)

    def phase(h, p):
        gather_desc(h, p).wait()
        transpose_chunk(p)
        scatter_desc(h, p).start()
        f = h + 2
        pf = (p + 2) % NGRP
        scatter_desc(f - NGRP, pf).wait()
        gather_desc(f, pf).start()

    gather_desc(0, 0).start()
    gather_desc(1, 1).start()

    gather_desc(0, 0).wait()
    transpose_chunk(0)
    scatter_desc(0, 0).start()
    gather_desc(2, 2).start()

    gather_desc(1, 1).wait()
    transpose_chunk(1)
    scatter_desc(1, 1).start()
    scatter_desc(0, 0).wait()
    gather_desc(3, 0).start()

    gather_desc(2, 2).wait()
    transpose_chunk(2)
    scatter_desc(2, 2).start()
    scatter_desc(1, 1).wait()
    gather_desc(4, 1).start()

    def body(i, c):
        h = 3 * i
        phase(h + 0, 0)
        phase(h + 1, 1)
        phase(h + 2, 2)
        return c

    lax.fori_loop(1, 16, body, 0)

    gather_desc(48, 0).wait()
    transpose_chunk(0)
    scatter_desc(48, 0).start()

    gather_desc(49, 1).wait()
    transpose_chunk(1)
    scatter_desc(49, 1).start()

    scatter_desc(47, 2).wait()
    scatter_desc(48, 0).wait()
    scatter_desc(49, 1).wait()


def kernel(x, table):
    if x.ndim > 1 and x.shape[-1] == 1:
        x = x[..., 0]
    trep = jnp.tile(table.astype(jnp.float32)[None], (NW, 1, 1))
    idx3 = x.astype(jnp.int32).reshape(NW, IPT, HIST).transpose(0, 2, 1)
    o5 = _emb_lookup(trep, idx3)
    return o5.transpose(2, 4, 0, 1, 3).reshape(BATCH, HIST, D)
